# baseline (speedup 1.0000x reference)
import jax
import jax.numpy as jnp
from jax import lax
from jax.experimental import pallas as pl
from jax.experimental.pallas import tpu as pltpu
from jax.experimental.pallas import tpu_sc as plsc

NC = 2
NS = 16
NW = NC * NS
C = 16
VL = 16


def _round_bf16_bits(u):
    rnd = lax.bitwise_and(lax.shift_right_logical(u, 16), 1) + 32767
    return lax.shift_right_logical(u + rnd, 16)


def _tc_pack(features):
    n, d = features.shape
    h = d // 2
    bn = next(c for c in (1024, 1000, 512, 400, 256, 200, 128, 100, 80, 64,
                          50, 40, 32, 25, 16, 8, 5, 4, 2, 1) if n % c == 0)

    def body(x_ref, o_ref):
        x = x_ref[...]
        lo = lax.bitcast_convert_type(x[:, :h], jnp.int32)
        hi = lax.bitcast_convert_type(x[:, h:], jnp.int32)
        lo16 = _round_bf16_bits(lo)
        hi16 = lax.shift_left(_round_bf16_bits(hi), 16)
        o_ref[...] = lax.bitwise_or(lo16, hi16)

    return pl.pallas_call(
        body,
        grid=(n // bn,),
        in_specs=[pl.BlockSpec((bn, d), lambda i: (i, 0))],
        out_specs=pl.BlockSpec((bn, h), lambda i: (i, 0)),
        out_shape=jax.ShapeDtypeStruct((n, h), jnp.int32),
    )(features)


def _sc_gather_mean(neigh_flat, nodes_p, feat_i32, b_per_w, s,
                    slab_base, b_slab):
    dw = feat_i32.shape[1]
    d = 2 * dw
    rows = C * s
    n_chunks = b_per_w // C
    nvec = dw // VL
    splits = []
    off = 0
    while off < rows:
        n = min(128, rows - off)
        splits.append((off, n))
        off += n

    mesh = plsc.VectorSubcoreMesh(core_axis_name="c", subcore_axis_name="s")

    def body(neigh_hbm, nodes_hbm, feat_hbm, self_out, agg_out,
             nidx0, nidx1, sidx0, sidx1, rows0, rows1, selfr0, selfr1,
             agg0, agg1, sem_n0, sem_n1, sem_s0, sem_s1, sem_i0, sem_i1,
             sem_o0, sem_o1, sem_a0, sem_a1):
        wid = lax.axis_index("s") * NC + lax.axis_index("c")
        base = wid * b_per_w
        nidx = (nidx0, nidx1)
        sidx = (sidx0, sidx1)
        rows_v = (rows0, rows1)
        selfr = (selfr0, selfr1)
        agg = (agg0, agg1)
        sem_n = (sem_n0, sem_n1)
        sem_s = (sem_s0, sem_s1)
        sem_i = (sem_i0, sem_i1)
        sem_o = (sem_o0, sem_o1)
        sem_a = (sem_a0, sem_a1)

        def stage_idx(ci, slot):
            gb = slab_base + base + ci * C
            pltpu.async_copy(neigh_hbm.at[pl.ds(gb * s, rows)], nidx[slot],
                             sem_i[slot])
            pltpu.async_copy(nodes_hbm.at[pl.ds(gb, C)], sidx[slot],
                             sem_i[slot])

        def stage_gather(slot):
            pltpu.make_async_copy(neigh_hbm.at[pl.ds(0, rows)],
                                  nidx[slot], sem_i[slot]).wait()
            pltpu.make_async_copy(nodes_hbm.at[pl.ds(0, C)],
                                  sidx[slot], sem_i[slot]).wait()
            for (o, n) in splits:
                pltpu.async_copy(feat_hbm.at[nidx[slot].at[pl.ds(o, n)]],
                                 rows_v[slot].at[pl.ds(o, n)], sem_n[slot])
            pltpu.async_copy(feat_hbm.at[sidx[slot]], selfr[slot],
                             sem_s[slot])

        def compute(slot):
            rv = rows_v[slot]
            av = agg[slot]

            def node(i, c2):
                def row(j, accs):
                    r = i * s + j
                    new = []
                    for v in range(nvec):
                        w = rv[r, pl.ds(v * VL, VL)]
                        lo = lax.bitcast_convert_type(
                            lax.shift_left(w, jnp.int32(16)), jnp.float32)
                        hi = lax.bitcast_convert_type(w, jnp.float32)
                        new.append(accs[2 * v] + lo)
                        new.append(accs[2 * v + 1] + hi)
                    return tuple(new)

                accs = lax.fori_loop(
                    0, s, row,
                    tuple(jnp.zeros((VL,), jnp.float32)
                          for _ in range(2 * nvec)))
                for v in range(nvec):
                    av[i, pl.ds(v * VL, VL)] = accs[2 * v]
                    av[i, pl.ds(dw + v * VL, VL)] = accs[2 * v + 1]
                return c2

            lax.fori_loop(0, C, node, 0)

        def maybe_when(cond, fn):
            if isinstance(cond, bool):
                if cond:
                    fn()
            else:
                pl.when(cond)(fn)

        def step(ci, slot, first):
            cb = base + ci * C
            pltpu.make_async_copy(feat_hbm.at[pl.ds(0, rows)],
                                  rows_v[slot], sem_n[slot]).wait()
            pltpu.make_async_copy(feat_hbm.at[pl.ds(0, C)],
                                  selfr[slot], sem_s[slot]).wait()
            pltpu.async_copy(selfr[slot], self_out.at[pl.ds(cb, C)],
                             sem_o[slot])

            maybe_when(ci + 2 < n_chunks, lambda: stage_idx(ci + 2, slot))

            if not first:
                pltpu.make_async_copy(agg[slot],
                                      agg_out.at[pl.ds(0, C)],
                                      sem_a[slot]).wait()

            compute(slot)
            pltpu.async_copy(agg[slot], agg_out.at[pl.ds(cb, C)],
                             sem_a[slot])
            pltpu.make_async_copy(selfr[slot],
                                  self_out.at[pl.ds(0, C)],
                                  sem_o[slot]).wait()

            maybe_when(ci + 2 < n_chunks, lambda: stage_gather(slot))

        stage_idx(0, 0)
        stage_idx(1, 1)
        stage_gather(0)
        stage_gather(1)
        step(0, 0, True)
        step(1, 1, True)

        def pair(g, carry):
            for slot in range(2):
                step(2 * g + slot, slot, False)
            return carry

        lax.fori_loop(1, n_chunks // 2, pair, 0)
        if n_chunks % 2:
            step(n_chunks - 1, 0, False)
        for slot in range(2):
            pltpu.make_async_copy(agg[slot], agg_out.at[pl.ds(0, C)],
                                  sem_a[slot]).wait()

    f = pl.kernel(
        body,
        out_type=(jax.ShapeDtypeStruct((b_slab, dw), jnp.int32),
                  jax.ShapeDtypeStruct((b_slab, d), jnp.float32)),
        mesh=mesh,
        scratch_types=[
            pltpu.VMEM((rows,), jnp.int32),
            pltpu.VMEM((rows,), jnp.int32),
            pltpu.VMEM((C,), jnp.int32),
            pltpu.VMEM((C,), jnp.int32),
            pltpu.VMEM((rows, dw), jnp.int32),
            pltpu.VMEM((rows, dw), jnp.int32),
            pltpu.VMEM((C, dw), jnp.int32),
            pltpu.VMEM((C, dw), jnp.int32),
            pltpu.VMEM((C, d), jnp.float32),
            pltpu.VMEM((C, d), jnp.float32),
        ] + [pltpu.SemaphoreType.DMA] * 10,
    )
    return f(neigh_flat, nodes_p, feat_i32)


def _tc_matmul_slab(selfs_i, aggs_f32, w_bf, b, blk_off, prev=None,
                    bt=512):
    b_slab, d = aggs_f32.shape
    dw = d // 2
    e = w_bf.shape[0]
    dims = (((1,), (1,)), ((), ()))
    n_blocks = b_slab // bt

    def body(*refs):
        if prev is None:
            self_ref, agg_ref, w_ref, out_ref = refs
        else:
            self_ref, agg_ref, w_ref, _, out_ref = refs
        si = self_ref[...]
        s_lo = lax.bitcast_convert_type(
            lax.shift_left(si, jnp.int32(16)), jnp.float32).astype(jnp.bfloat16)
        s_hi = lax.bitcast_convert_type(
            lax.bitwise_and(si, jnp.int32(-65536)),
            jnp.float32).astype(jnp.bfloat16)
        a_bf = agg_ref[...].astype(jnp.bfloat16)
        w = w_ref[...]
        acc = lax.dot_general(w[:, :dw], s_lo, dims,
                              preferred_element_type=jnp.float32)
        acc += lax.dot_general(w[:, dw:d], s_hi, dims,
                               preferred_element_type=jnp.float32)
        acc += lax.dot_general(w[:, d:], a_bf, dims,
                               preferred_element_type=jnp.float32)
        out_ref[...] = jnp.maximum(acc, 0.0)

    in_specs = [
        pl.BlockSpec((bt, dw), lambda i: (i, 0)),
        pl.BlockSpec((bt, d), lambda i: (i, 0)),
        pl.BlockSpec((e, 2 * d), lambda i: (0, 0)),
    ]
    args = [selfs_i, aggs_f32, w_bf]
    kwargs = {}
    if prev is not None:
        in_specs.append(pl.BlockSpec(memory_space=pl.ANY))
        args.append(prev)
        kwargs["input_output_aliases"] = {3: 0}
    return pl.pallas_call(
        body,
        grid=(n_blocks,),
        in_specs=in_specs,
        out_specs=pl.BlockSpec((e, bt), lambda i: (0, i + blk_off)),
        out_shape=jax.ShapeDtypeStruct((e, b), jnp.float32),
        **kwargs,
    )(*args)


def kernel(nodes, neigh_idx, features, weight):
    b = nodes.shape[0]
    n, d = features.shape
    s = neigh_idx.shape[1]

    quantum = NW * C
    b_pad = -(-b // quantum) * quantum
    pad = b_pad - b
    if pad:
        pad_nodes = (jnp.arange(pad, dtype=jnp.int32) * 97) % n
        nodes_p = jnp.concatenate([nodes, pad_nodes])
        pad_neigh = ((jnp.arange(pad * s, dtype=jnp.int32) * 131) % n)
        neigh_p = jnp.concatenate([neigh_idx.reshape(-1), pad_neigh])
    else:
        nodes_p = nodes
        neigh_p = neigh_idx.reshape(-1)

    feat_i32 = _tc_pack(features)
    w_bf = jnp.concatenate(
        [weight[:, :d], weight[:, d:] * jnp.float32(1.0 / s)],
        axis=1).astype(jnp.bfloat16)

    n_slabs = 2
    bt = 512
    b_slab = b_pad // n_slabs
    slab_results = []
    for k in range(n_slabs):
        slab_results.append(
            _sc_gather_mean(neigh_p, nodes_p, feat_i32, b_slab // NW, s,
                            k * b_slab, b_slab))
    out = None
    for k in range(n_slabs):
        selfs_i, aggs = slab_results[k]
        out = _tc_matmul_slab(selfs_i, aggs, w_bf, b,
                              k * (b_slab // bt), prev=out, bt=bt)
    return out


# speedup vs baseline: 1.0211x; 1.0211x over previous
import jax
import jax.numpy as jnp
from jax import lax
from jax.experimental import pallas as pl
from jax.experimental.pallas import tpu as pltpu
from jax.experimental.pallas import tpu_sc as plsc

NC = 2
NS = 16
NW = NC * NS
C = 16
VL = 16


def _round_bf16_bits(u):
    rnd = lax.bitwise_and(lax.shift_right_logical(u, 16), 1) + 32767
    return lax.shift_right_logical(u + rnd, 16)


def _tc_pack(features):
    n, d = features.shape
    h = d // 2
    bn = next(c for c in (1024, 1000, 512, 400, 256, 200, 128, 100, 80, 64,
                          50, 40, 32, 25, 16, 8, 5, 4, 2, 1) if n % c == 0)

    def body(x_ref, o_ref):
        x = x_ref[...]
        lo = lax.bitcast_convert_type(x[:, :h], jnp.int32)
        hi = lax.bitcast_convert_type(x[:, h:], jnp.int32)
        lo16 = _round_bf16_bits(lo)
        hi16 = lax.shift_left(_round_bf16_bits(hi), 16)
        o_ref[...] = lax.bitwise_or(lo16, hi16)

    return pl.pallas_call(
        body,
        grid=(n // bn,),
        in_specs=[pl.BlockSpec((bn, d), lambda i: (i, 0))],
        out_specs=pl.BlockSpec((bn, h), lambda i: (i, 0)),
        out_shape=jax.ShapeDtypeStruct((n, h), jnp.int32),
    )(features)


def _sc_gather_mean(neigh_flat, nodes_p, feat_i32, b_per_w, s):
    b_pad = nodes_p.shape[0]
    dw = feat_i32.shape[1]
    d = 2 * dw
    rows = C * s
    n_chunks = b_per_w // C
    nvec = dw // VL
    splits = []
    off = 0
    while off < rows:
        n = min(128, rows - off)
        splits.append((off, n))
        off += n

    mesh = plsc.VectorSubcoreMesh(core_axis_name="c", subcore_axis_name="s")

    def body(neigh_hbm, nodes_hbm, feat_hbm, self_out, agg_out,
             nidx0, nidx1, sidx0, sidx1, rows0, rows1, selfr0, selfr1,
             agg0, agg1, sem_n0, sem_n1, sem_s0, sem_s1, sem_i0, sem_i1,
             sem_o0, sem_o1, sem_a0, sem_a1):
        wid = lax.axis_index("s") * NC + lax.axis_index("c")
        base = wid * b_per_w
        nidx = (nidx0, nidx1)
        sidx = (sidx0, sidx1)
        rows_v = (rows0, rows1)
        selfr = (selfr0, selfr1)
        agg = (agg0, agg1)
        sem_n = (sem_n0, sem_n1)
        sem_s = (sem_s0, sem_s1)
        sem_i = (sem_i0, sem_i1)
        sem_o = (sem_o0, sem_o1)
        sem_a = (sem_a0, sem_a1)

        def stage_idx(ci, slot):
            cb = base + ci * C
            pltpu.async_copy(neigh_hbm.at[pl.ds(cb * s, rows)], nidx[slot],
                             sem_i[slot])
            pltpu.async_copy(nodes_hbm.at[pl.ds(cb, C)], sidx[slot],
                             sem_i[slot])

        def stage_gather(slot):
            pltpu.make_async_copy(neigh_hbm.at[pl.ds(0, rows)],
                                  nidx[slot], sem_i[slot]).wait()
            pltpu.make_async_copy(nodes_hbm.at[pl.ds(0, C)],
                                  sidx[slot], sem_i[slot]).wait()
            for (o, n) in splits:
                pltpu.async_copy(feat_hbm.at[nidx[slot].at[pl.ds(o, n)]],
                                 rows_v[slot].at[pl.ds(o, n)], sem_n[slot])
            pltpu.async_copy(feat_hbm.at[sidx[slot]], selfr[slot],
                             sem_s[slot])

        def compute(slot):
            rv = rows_v[slot]
            av = agg[slot]

            def node(i, c2):
                accs = [jnp.zeros((VL,), jnp.float32)
                        for _ in range(2 * nvec)]
                for j in range(s):
                    r = i * s + j
                    for v in range(nvec):
                        w = rv[r, pl.ds(v * VL, VL)]
                        lo = lax.bitcast_convert_type(
                            lax.shift_left(w, jnp.int32(16)), jnp.float32)
                        hi = lax.bitcast_convert_type(w, jnp.float32)
                        accs[2 * v] = accs[2 * v] + lo
                        accs[2 * v + 1] = accs[2 * v + 1] + hi
                for v in range(nvec):
                    av[i, pl.ds(v * VL, VL)] = accs[2 * v]
                    av[i, pl.ds(dw + v * VL, VL)] = accs[2 * v + 1]
                return c2

            lax.fori_loop(0, C, node, 0)

        stage_idx(0, 0)
        stage_idx(1, 1)
        stage_gather(0)
        stage_gather(1)

        def pair(g, carry):
            for slot in range(2):
                ci = 2 * g + slot
                cb = base + ci * C
                pltpu.make_async_copy(feat_hbm.at[pl.ds(0, rows)],
                                      rows_v[slot], sem_n[slot]).wait()
                pltpu.make_async_copy(feat_hbm.at[pl.ds(0, C)],
                                      selfr[slot], sem_s[slot]).wait()
                pltpu.async_copy(selfr[slot], self_out.at[pl.ds(cb, C)],
                                 sem_o[slot])

                @pl.when(ci + 2 < n_chunks)
                def _():
                    stage_idx(ci + 2, slot)

                @pl.when(ci >= 2)
                def _():
                    pltpu.make_async_copy(agg[slot],
                                          agg_out.at[pl.ds(0, C)],
                                          sem_a[slot]).wait()

                compute(slot)
                pltpu.async_copy(agg[slot], agg_out.at[pl.ds(cb, C)],
                                 sem_a[slot])
                pltpu.make_async_copy(selfr[slot],
                                      self_out.at[pl.ds(0, C)],
                                      sem_o[slot]).wait()

                @pl.when(ci + 2 < n_chunks)
                def _():
                    stage_gather(slot)
            return carry

        lax.fori_loop(0, n_chunks // 2, pair, 0)
        for slot in range(2):
            pltpu.make_async_copy(agg[slot], agg_out.at[pl.ds(0, C)],
                                  sem_a[slot]).wait()

    f = pl.kernel(
        body,
        out_type=(jax.ShapeDtypeStruct((b_pad, dw), jnp.int32),
                  jax.ShapeDtypeStruct((b_pad, d), jnp.float32)),
        mesh=mesh,
        scratch_types=[
            pltpu.VMEM((rows,), jnp.int32),
            pltpu.VMEM((rows,), jnp.int32),
            pltpu.VMEM((C,), jnp.int32),
            pltpu.VMEM((C,), jnp.int32),
            pltpu.VMEM((rows, dw), jnp.int32),
            pltpu.VMEM((rows, dw), jnp.int32),
            pltpu.VMEM((C, dw), jnp.int32),
            pltpu.VMEM((C, dw), jnp.int32),
            pltpu.VMEM((C, d), jnp.float32),
            pltpu.VMEM((C, d), jnp.float32),
        ] + [pltpu.SemaphoreType.DMA] * 10,
    )
    return f(neigh_flat, nodes_p, feat_i32)


def _tc_matmul(selfs_i, aggs_f32, w_bf, b, bt=1024):
    b_pad, d = aggs_f32.shape
    dw = d // 2
    e = w_bf.shape[0]
    dims = (((1,), (1,)), ((), ()))

    def body(self_ref, agg_ref, w_ref, out_ref):
        si = self_ref[...]
        s_lo = lax.bitcast_convert_type(
            lax.shift_left(si, jnp.int32(16)), jnp.float32).astype(jnp.bfloat16)
        s_hi = lax.bitcast_convert_type(
            lax.bitwise_and(si, jnp.int32(-65536)),
            jnp.float32).astype(jnp.bfloat16)
        a_bf = agg_ref[...].astype(jnp.bfloat16)
        w = w_ref[...]
        acc = lax.dot_general(w[:, :dw], s_lo, dims,
                              preferred_element_type=jnp.float32)
        acc += lax.dot_general(w[:, dw:d], s_hi, dims,
                               preferred_element_type=jnp.float32)
        acc += lax.dot_general(w[:, d:], a_bf, dims,
                               preferred_element_type=jnp.float32)
        out_ref[...] = jnp.maximum(acc, 0.0)

    return pl.pallas_call(
        body,
        grid=(b_pad // bt,),
        in_specs=[
            pl.BlockSpec((bt, dw), lambda i: (i, 0)),
            pl.BlockSpec((bt, d), lambda i: (i, 0)),
            pl.BlockSpec((e, 2 * d), lambda i: (0, 0)),
        ],
        out_specs=pl.BlockSpec((e, bt), lambda i: (0, i)),
        out_shape=jax.ShapeDtypeStruct((e, b), jnp.float32),
    )(selfs_i, aggs_f32, w_bf)


def kernel(nodes, neigh_idx, features, weight):
    b = nodes.shape[0]
    n, d = features.shape
    s = neigh_idx.shape[1]

    quantum = NW * C
    b_pad = -(-b // quantum) * quantum
    pad = b_pad - b
    if pad:
        pad_nodes = (jnp.arange(pad, dtype=jnp.int32) * 97) % n
        nodes_p = jnp.concatenate([nodes, pad_nodes])
        pad_neigh = ((jnp.arange(pad * s, dtype=jnp.int32) * 131) % n)
        neigh_p = jnp.concatenate([neigh_idx.reshape(-1), pad_neigh])
    else:
        nodes_p = nodes
        neigh_p = neigh_idx.reshape(-1)

    feat_i32 = _tc_pack(features)
    selfs_i, aggs = _sc_gather_mean(neigh_p, nodes_p, feat_i32,
                                    b_pad // NW, s)
    w_bf = jnp.concatenate(
        [weight[:, :d], weight[:, d:] * jnp.float32(1.0 / s)],
        axis=1).astype(jnp.bfloat16)
    return _tc_matmul(selfs_i, aggs, w_bf, b)


# speedup vs baseline: 1.0224x; 1.0012x over previous
import jax
import jax.numpy as jnp
from jax import lax
from jax.experimental import pallas as pl
from jax.experimental.pallas import tpu as pltpu
from jax.experimental.pallas import tpu_sc as plsc

NC = 2
NS = 16
NW = NC * NS
C = 16
VL = 16


def _round_bf16_bits(u):
    rnd = lax.bitwise_and(lax.shift_right_logical(u, 16), 1) + 32767
    return lax.shift_right_logical(u + rnd, 16)


def _tc_pack(features):
    n, d = features.shape
    h = d // 2
    bn = next(c for c in (1024, 1000, 512, 400, 256, 200, 128, 100, 80, 64,
                          50, 40, 32, 25, 16, 8, 5, 4, 2, 1) if n % c == 0)

    def body(x_ref, o_ref):
        x = x_ref[...]
        lo = lax.bitcast_convert_type(x[:, :h], jnp.int32)
        hi = lax.bitcast_convert_type(x[:, h:], jnp.int32)
        lo16 = _round_bf16_bits(lo)
        hi16 = lax.shift_left(_round_bf16_bits(hi), 16)
        o_ref[...] = lax.bitwise_or(lo16, hi16)

    return pl.pallas_call(
        body,
        grid=(n // bn,),
        in_specs=[pl.BlockSpec((bn, d), lambda i: (i, 0))],
        out_specs=pl.BlockSpec((bn, h), lambda i: (i, 0)),
        out_shape=jax.ShapeDtypeStruct((n, h), jnp.int32),
    )(features)


def _sc_gather_mean(neigh_flat, nodes_p, feat_i32, b_per_w, s):
    b_pad = nodes_p.shape[0]
    dw = feat_i32.shape[1]
    d = 2 * dw
    rows = C * s
    n_chunks = b_per_w // C
    nvec = dw // VL
    splits = []
    off = 0
    while off < rows:
        n = min(128, rows - off)
        splits.append((off, n))
        off += n

    mesh = plsc.VectorSubcoreMesh(core_axis_name="c", subcore_axis_name="s")

    def body(neigh_hbm, nodes_hbm, feat_hbm, self_out, agg_out,
             nidx0, nidx1, sidx0, sidx1, rows0, rows1, selfr0, selfr1,
             agg0, agg1, sem_n0, sem_n1, sem_s0, sem_s1, sem_i0, sem_i1,
             sem_o0, sem_o1, sem_a0, sem_a1):
        wid = lax.axis_index("s") * NC + lax.axis_index("c")
        base = wid * b_per_w
        nidx = (nidx0, nidx1)
        sidx = (sidx0, sidx1)
        rows_v = (rows0, rows1)
        selfr = (selfr0, selfr1)
        agg = (agg0, agg1)
        sem_n = (sem_n0, sem_n1)
        sem_s = (sem_s0, sem_s1)
        sem_i = (sem_i0, sem_i1)
        sem_o = (sem_o0, sem_o1)
        sem_a = (sem_a0, sem_a1)

        def stage_idx(ci, slot):
            cb = base + ci * C
            pltpu.async_copy(neigh_hbm.at[pl.ds(cb * s, rows)], nidx[slot],
                             sem_i[slot])
            pltpu.async_copy(nodes_hbm.at[pl.ds(cb, C)], sidx[slot],
                             sem_i[slot])

        def stage_gather(slot):
            pltpu.make_async_copy(neigh_hbm.at[pl.ds(0, rows)],
                                  nidx[slot], sem_i[slot]).wait()
            pltpu.make_async_copy(nodes_hbm.at[pl.ds(0, C)],
                                  sidx[slot], sem_i[slot]).wait()
            for (o, n) in splits:
                pltpu.async_copy(feat_hbm.at[nidx[slot].at[pl.ds(o, n)]],
                                 rows_v[slot].at[pl.ds(o, n)], sem_n[slot])
            pltpu.async_copy(feat_hbm.at[sidx[slot]], selfr[slot],
                             sem_s[slot])

        hi_mask = jnp.full((VL,), -65536, dtype=jnp.int32)

        def compute(slot):
            rv = rows_v[slot]
            av = agg[slot]

            def node(i, c2):
                accs = [jnp.zeros((VL,), jnp.float32)
                        for _ in range(2 * nvec)]
                for j in range(s):
                    r = i * s + j
                    for v in range(nvec):
                        w = rv[r, pl.ds(v * VL, VL)]
                        lo = lax.bitcast_convert_type(
                            lax.shift_left(w, jnp.int32(16)), jnp.float32)
                        hi = lax.bitcast_convert_type(w, jnp.float32)
                        accs[2 * v] = accs[2 * v] + lo
                        accs[2 * v + 1] = accs[2 * v + 1] + hi
                for v in range(nvec):
                    ulo = lax.bitcast_convert_type(accs[2 * v], jnp.int32)
                    uhi = lax.bitcast_convert_type(accs[2 * v + 1], jnp.int32)
                    rlo = lax.shift_right_logical(
                        ulo + (lax.bitwise_and(
                            lax.shift_right_logical(ulo, 16), 1) + 32767), 16)
                    rhi = lax.bitwise_and(
                        uhi + (lax.bitwise_and(
                            lax.shift_right_logical(uhi, 16), 1) + 32767),
                        hi_mask)
                    av[i, pl.ds(v * VL, VL)] = lax.bitwise_or(rlo, rhi)
                return c2

            lax.fori_loop(0, C, node, 0)

        stage_idx(0, 0)
        stage_idx(1, 1)
        stage_gather(0)
        stage_gather(1)

        def pair(g, carry):
            for slot in range(2):
                ci = 2 * g + slot
                cb = base + ci * C
                pltpu.make_async_copy(feat_hbm.at[pl.ds(0, rows)],
                                      rows_v[slot], sem_n[slot]).wait()
                pltpu.make_async_copy(feat_hbm.at[pl.ds(0, C)],
                                      selfr[slot], sem_s[slot]).wait()
                pltpu.async_copy(selfr[slot], self_out.at[pl.ds(cb, C)],
                                 sem_o[slot])

                @pl.when(ci + 2 < n_chunks)
                def _():
                    stage_idx(ci + 2, slot)

                @pl.when(ci >= 2)
                def _():
                    pltpu.make_async_copy(agg[slot],
                                          agg_out.at[pl.ds(0, C)],
                                          sem_a[slot]).wait()

                compute(slot)
                pltpu.async_copy(agg[slot], agg_out.at[pl.ds(cb, C)],
                                 sem_a[slot])
                pltpu.make_async_copy(selfr[slot],
                                      self_out.at[pl.ds(0, C)],
                                      sem_o[slot]).wait()

                @pl.when(ci + 2 < n_chunks)
                def _():
                    stage_gather(slot)
            return carry

        lax.fori_loop(0, n_chunks // 2, pair, 0)
        for slot in range(2):
            pltpu.make_async_copy(agg[slot], agg_out.at[pl.ds(0, C)],
                                  sem_a[slot]).wait()

    f = pl.kernel(
        body,
        out_type=(jax.ShapeDtypeStruct((b_pad, dw), jnp.int32),
                  jax.ShapeDtypeStruct((b_pad, dw), jnp.int32)),
        mesh=mesh,
        scratch_types=[
            pltpu.VMEM((rows,), jnp.int32),
            pltpu.VMEM((rows,), jnp.int32),
            pltpu.VMEM((C,), jnp.int32),
            pltpu.VMEM((C,), jnp.int32),
            pltpu.VMEM((rows, dw), jnp.int32),
            pltpu.VMEM((rows, dw), jnp.int32),
            pltpu.VMEM((C, dw), jnp.int32),
            pltpu.VMEM((C, dw), jnp.int32),
            pltpu.VMEM((C, dw), jnp.int32),
            pltpu.VMEM((C, dw), jnp.int32),
        ] + [pltpu.SemaphoreType.DMA] * 10,
    )
    return f(neigh_flat, nodes_p, feat_i32)


def _tc_matmul(selfs_i, aggs_i, w_bf, b, bt=1024):
    b_pad, dw = aggs_i.shape
    d = 2 * dw
    e = w_bf.shape[0]
    dims = (((1,), (1,)), ((), ()))

    def unpack(x):
        lo = lax.bitcast_convert_type(
            lax.shift_left(x, jnp.int32(16)), jnp.float32).astype(jnp.bfloat16)
        hi = lax.bitcast_convert_type(
            lax.bitwise_and(x, jnp.int32(-65536)),
            jnp.float32).astype(jnp.bfloat16)
        return lo, hi

    def body(self_ref, agg_ref, w_ref, out_ref):
        s_lo, s_hi = unpack(self_ref[...])
        a_lo, a_hi = unpack(agg_ref[...])
        w = w_ref[...]
        acc = lax.dot_general(w[:, :dw], s_lo, dims,
                              preferred_element_type=jnp.float32)
        acc += lax.dot_general(w[:, dw:d], s_hi, dims,
                               preferred_element_type=jnp.float32)
        acc += lax.dot_general(w[:, d:d + dw], a_lo, dims,
                               preferred_element_type=jnp.float32)
        acc += lax.dot_general(w[:, d + dw:], a_hi, dims,
                               preferred_element_type=jnp.float32)
        out_ref[...] = jnp.maximum(acc, 0.0)

    return pl.pallas_call(
        body,
        grid=(b_pad // bt,),
        in_specs=[
            pl.BlockSpec((bt, dw), lambda i: (i, 0)),
            pl.BlockSpec((bt, dw), lambda i: (i, 0)),
            pl.BlockSpec((e, 2 * d), lambda i: (0, 0)),
        ],
        out_specs=pl.BlockSpec((e, bt), lambda i: (0, i)),
        out_shape=jax.ShapeDtypeStruct((e, b), jnp.float32),
    )(selfs_i, aggs_i, w_bf)


def kernel(nodes, neigh_idx, features, weight):
    b = nodes.shape[0]
    n, d = features.shape
    s = neigh_idx.shape[1]

    quantum = NW * C
    b_pad = -(-b // quantum) * quantum
    pad = b_pad - b
    if pad:
        pad_nodes = (jnp.arange(pad, dtype=jnp.int32) * 97) % n
        nodes_p = jnp.concatenate([nodes, pad_nodes])
        pad_neigh = ((jnp.arange(pad * s, dtype=jnp.int32) * 131) % n)
        neigh_p = jnp.concatenate([neigh_idx.reshape(-1), pad_neigh])
    else:
        nodes_p = nodes
        neigh_p = neigh_idx.reshape(-1)

    feat_i32 = _tc_pack(features)
    selfs_i, aggs = _sc_gather_mean(neigh_p, nodes_p, feat_i32,
                                    b_pad // NW, s)
    w_bf = jnp.concatenate(
        [weight[:, :d], weight[:, d:] * jnp.float32(1.0 / s)],
        axis=1).astype(jnp.bfloat16)
    return _tc_matmul(selfs_i, aggs, w_bf, b)


# speedup vs baseline: 1.0302x; 1.0077x over previous
import jax
import jax.numpy as jnp
from jax import lax
from jax.experimental import pallas as pl
from jax.experimental.pallas import tpu as pltpu
from jax.experimental.pallas import tpu_sc as plsc

NC = 2
NS = 16
NW = NC * NS
C = 16
VL = 16


def _round_bf16_bits(u):
    rnd = lax.bitwise_and(lax.shift_right_logical(u, 16), 1) + 32767
    return lax.shift_right_logical(u + rnd, 16)


def _tc_pack(features):
    n, d = features.shape
    h = d // 2
    bn = next(c for c in (1024, 1000, 512, 400, 256, 200, 128, 100, 80, 64,
                          50, 40, 32, 25, 16, 8, 5, 4, 2, 1) if n % c == 0)

    def body(x_ref, o_ref):
        x = x_ref[...]
        lo = lax.bitcast_convert_type(x[:, :h], jnp.int32)
        hi = lax.bitcast_convert_type(x[:, h:], jnp.int32)
        lo16 = _round_bf16_bits(lo)
        hi16 = lax.shift_left(_round_bf16_bits(hi), 16)
        o_ref[...] = lax.bitwise_or(lo16, hi16)

    return pl.pallas_call(
        body,
        grid=(n // bn,),
        in_specs=[pl.BlockSpec((bn, d), lambda i: (i, 0))],
        out_specs=pl.BlockSpec((bn, h), lambda i: (i, 0)),
        out_shape=jax.ShapeDtypeStruct((n, h), jnp.int32),
    )(features)


def _sc_gather_mean(neigh_flat, nodes_p, feat_i32, b_per_w, s, b_real):
    b_pad = nodes_p.shape[0]
    dw = feat_i32.shape[1]
    d = 2 * dw
    rows = C * s
    n_chunks = b_per_w // C
    nvec = dw // VL
    splits = []
    off = 0
    while off < rows:
        n = min(128, rows - off)
        splits.append((off, n))
        off += n

    mesh = plsc.VectorSubcoreMesh(core_axis_name="c", subcore_axis_name="s")

    def body(neigh_hbm, nodes_hbm, feat_hbm, self_out, agg_out,
             nidx0, nidx1, sidx0, sidx1, rows0, rows1, selfr0, selfr1,
             agg0, agg1, sem_n0, sem_n1, sem_s0, sem_s1, sem_i0, sem_i1,
             sem_o0, sem_o1, sem_a0, sem_a1):
        wid = lax.axis_index("s") * NC + lax.axis_index("c")
        base = wid * b_per_w
        nidx = (nidx0, nidx1)
        sidx = (sidx0, sidx1)
        rows_v = (rows0, rows1)
        selfr = (selfr0, selfr1)
        agg = (agg0, agg1)
        sem_n = (sem_n0, sem_n1)
        sem_s = (sem_s0, sem_s1)
        sem_i = (sem_i0, sem_i1)
        sem_o = (sem_o0, sem_o1)
        sem_a = (sem_a0, sem_a1)

        def stage_idx(ci, slot):
            cb = jnp.minimum(base + ci * C, b_real - C)
            pltpu.async_copy(neigh_hbm.at[pl.ds(cb * s, rows)], nidx[slot],
                             sem_i[slot])
            pltpu.async_copy(nodes_hbm.at[pl.ds(cb, C)], sidx[slot],
                             sem_i[slot])

        def stage_gather(slot):
            pltpu.make_async_copy(neigh_hbm.at[pl.ds(0, rows)],
                                  nidx[slot], sem_i[slot]).wait()
            pltpu.make_async_copy(nodes_hbm.at[pl.ds(0, C)],
                                  sidx[slot], sem_i[slot]).wait()
            for (o, n) in splits:
                pltpu.async_copy(feat_hbm.at[nidx[slot].at[pl.ds(o, n)]],
                                 rows_v[slot].at[pl.ds(o, n)], sem_n[slot])
            pltpu.async_copy(feat_hbm.at[sidx[slot]], selfr[slot],
                             sem_s[slot])

        hi_mask = jnp.full((VL,), -65536, dtype=jnp.int32)

        def compute(slot):
            rv = rows_v[slot]
            av = agg[slot]

            def node(i, c2):
                accs = [jnp.zeros((VL,), jnp.float32)
                        for _ in range(2 * nvec)]
                for j in range(s):
                    r = i * s + j
                    for v in range(nvec):
                        w = rv[r, pl.ds(v * VL, VL)]
                        lo = lax.bitcast_convert_type(
                            lax.shift_left(w, jnp.int32(16)), jnp.float32)
                        hi = lax.bitcast_convert_type(w, jnp.float32)
                        accs[2 * v] = accs[2 * v] + lo
                        accs[2 * v + 1] = accs[2 * v + 1] + hi
                for v in range(nvec):
                    ulo = lax.bitcast_convert_type(accs[2 * v], jnp.int32)
                    uhi = lax.bitcast_convert_type(accs[2 * v + 1], jnp.int32)
                    rlo = lax.shift_right_logical(
                        ulo + (lax.bitwise_and(
                            lax.shift_right_logical(ulo, 16), 1) + 32767), 16)
                    rhi = lax.bitwise_and(
                        uhi + (lax.bitwise_and(
                            lax.shift_right_logical(uhi, 16), 1) + 32767),
                        hi_mask)
                    av[i, pl.ds(v * VL, VL)] = lax.bitwise_or(rlo, rhi)
                return c2

            lax.fori_loop(0, C, node, 0)

        stage_idx(0, 0)
        stage_idx(1, 1)
        stage_gather(0)
        stage_gather(1)

        def pair(g, carry):
            for slot in range(2):
                ci = 2 * g + slot
                cb = base + ci * C
                pltpu.make_async_copy(feat_hbm.at[pl.ds(0, rows)],
                                      rows_v[slot], sem_n[slot]).wait()
                pltpu.make_async_copy(feat_hbm.at[pl.ds(0, C)],
                                      selfr[slot], sem_s[slot]).wait()
                pltpu.async_copy(selfr[slot], self_out.at[pl.ds(cb, C)],
                                 sem_o[slot])

                @pl.when(ci + 2 < n_chunks)
                def _():
                    stage_idx(ci + 2, slot)

                @pl.when(ci >= 2)
                def _():
                    pltpu.make_async_copy(agg[slot],
                                          agg_out.at[pl.ds(0, C)],
                                          sem_a[slot]).wait()

                compute(slot)
                pltpu.async_copy(agg[slot], agg_out.at[pl.ds(cb, C)],
                                 sem_a[slot])
                pltpu.make_async_copy(selfr[slot],
                                      self_out.at[pl.ds(0, C)],
                                      sem_o[slot]).wait()

                @pl.when(ci + 2 < n_chunks)
                def _():
                    stage_gather(slot)
            return carry

        lax.fori_loop(0, n_chunks // 2, pair, 0)
        for slot in range(2):
            pltpu.make_async_copy(agg[slot], agg_out.at[pl.ds(0, C)],
                                  sem_a[slot]).wait()

    f = pl.kernel(
        body,
        out_type=(jax.ShapeDtypeStruct((b_pad, dw), jnp.int32),
                  jax.ShapeDtypeStruct((b_pad, dw), jnp.int32)),
        mesh=mesh,
        scratch_types=[
            pltpu.VMEM((rows,), jnp.int32),
            pltpu.VMEM((rows,), jnp.int32),
            pltpu.VMEM((C,), jnp.int32),
            pltpu.VMEM((C,), jnp.int32),
            pltpu.VMEM((rows, dw), jnp.int32),
            pltpu.VMEM((rows, dw), jnp.int32),
            pltpu.VMEM((C, dw), jnp.int32),
            pltpu.VMEM((C, dw), jnp.int32),
            pltpu.VMEM((C, dw), jnp.int32),
            pltpu.VMEM((C, dw), jnp.int32),
        ] + [pltpu.SemaphoreType.DMA] * 10,
    )
    return f(neigh_flat, nodes_p, feat_i32)


def _tc_matmul(selfs_i, aggs_i, w_bf, b, bt=1024):
    b_pad, dw = aggs_i.shape
    d = 2 * dw
    e = w_bf.shape[0]
    dims = (((1,), (1,)), ((), ()))

    def unpack(x):
        lo = lax.bitcast_convert_type(
            lax.shift_left(x, jnp.int32(16)), jnp.float32).astype(jnp.bfloat16)
        hi = lax.bitcast_convert_type(
            lax.bitwise_and(x, jnp.int32(-65536)),
            jnp.float32).astype(jnp.bfloat16)
        return lo, hi

    def body(self_ref, agg_ref, w_ref, out_ref):
        s_lo, s_hi = unpack(self_ref[...])
        a_lo, a_hi = unpack(agg_ref[...])
        w = w_ref[...]
        acc = lax.dot_general(w[:, :dw], s_lo, dims,
                              preferred_element_type=jnp.float32)
        acc += lax.dot_general(w[:, dw:d], s_hi, dims,
                               preferred_element_type=jnp.float32)
        acc += lax.dot_general(w[:, d:d + dw], a_lo, dims,
                               preferred_element_type=jnp.float32)
        acc += lax.dot_general(w[:, d + dw:], a_hi, dims,
                               preferred_element_type=jnp.float32)
        out_ref[...] = jnp.maximum(acc, 0.0)

    return pl.pallas_call(
        body,
        grid=(b_pad // bt,),
        in_specs=[
            pl.BlockSpec((bt, dw), lambda i: (i, 0)),
            pl.BlockSpec((bt, dw), lambda i: (i, 0)),
            pl.BlockSpec((e, 2 * d), lambda i: (0, 0)),
        ],
        out_specs=pl.BlockSpec((e, bt), lambda i: (0, i)),
        out_shape=jax.ShapeDtypeStruct((e, b), jnp.float32),
    )(selfs_i, aggs_i, w_bf)


def kernel(nodes, neigh_idx, features, weight):
    b = nodes.shape[0]
    n, d = features.shape
    s = neigh_idx.shape[1]

    quantum = NW * C
    b_pad = -(-b // quantum) * quantum

    feat_i32 = _tc_pack(features)
    selfs_i, aggs = _sc_gather_mean(neigh_idx.reshape(-1), nodes, feat_i32,
                                    b_pad // NW, s, b)
    w_bf = jnp.concatenate(
        [weight[:, :d], weight[:, d:] * jnp.float32(1.0 / s)],
        axis=1).astype(jnp.bfloat16)
    return _tc_matmul(selfs_i, aggs, w_bf, b)
